# baseline (speedup 1.0000x reference)
import jax
import jax.numpy as jnp
import numpy as np
from jax.experimental import pallas as pl
from jax.experimental.pallas import tpu as pltpu

B = 32
H = 16
KV_LEN = 4096
KV_LORA_RANK = 512
QK_ROPE_HEAD_DIM = 64
V_HEAD_DIM = 128
D_MODEL = 4096
D_LAT = KV_LORA_RANK + QK_ROPE_HEAD_DIM
SCALE = 1.0 / np.sqrt(128.0 + 64.0)

B_BLK = 2
N_COL_CHUNKS = 2
COL_CHUNK = D_MODEL // N_COL_CHUNKS


def _attn_kernel(qn_ref, qp_ref, lat_ref, rope_ref, o_lat_ref):
    qn = qn_ref[...].astype(jnp.bfloat16)
    qp = qp_ref[...].astype(jnp.bfloat16)
    lat = lat_ref[...].astype(jnp.bfloat16)
    rope = rope_ref[...].astype(jnp.bfloat16)

    s = jax.lax.dot_general(
        qn, lat, (((2,), (1,)), ((0,), (0,))),
        preferred_element_type=jnp.float32,
    ) + jax.lax.dot_general(
        qp, rope, (((2,), (1,)), ((0,), (0,))),
        preferred_element_type=jnp.float32,
    )
    s = s * SCALE
    m = jnp.max(s, axis=-1, keepdims=True)
    p_f32 = jnp.exp(s - m)
    p = p_f32.astype(jnp.bfloat16)
    denom = jnp.sum(p_f32, axis=-1, keepdims=True)

    o_lat_ref[...] = jax.lax.dot_general(
        p, lat, (((2,), (2,)), ((0,), (0,))),
        preferred_element_type=jnp.float32,
    ) / denom


def _proj_kernel(o_lat_ref, w_uv_ref, w_o_ref, out_ref):
    o = jax.lax.dot_general(
        o_lat_ref[...].astype(jnp.bfloat16),
        w_uv_ref[...].astype(jnp.bfloat16),
        (((2,), (1,)), ((1,), (0,))),
        preferred_element_type=jnp.float32,
    )
    o = o.transpose(1, 0, 2).reshape(B, H * V_HEAD_DIM).astype(jnp.bfloat16)
    out_ref[...] = jax.lax.dot_general(
        o, w_o_ref[...].astype(jnp.bfloat16),
        (((1,), (0,)), ((), ())),
        preferred_element_type=jnp.float32,
    )


@jax.jit
def kernel(q_nope, q_pe, kv_cache, w_uv, w_o):
    kv_t = jnp.transpose(kv_cache, (0, 2, 1))

    o_lat = pl.pallas_call(
        _attn_kernel,
        grid=(B // B_BLK,),
        in_specs=[
            pl.BlockSpec((B_BLK, H, KV_LORA_RANK), lambda b: (b, 0, 0)),
            pl.BlockSpec((B_BLK, H, QK_ROPE_HEAD_DIM), lambda b: (b, 0, 0)),
            pl.BlockSpec((B_BLK, KV_LORA_RANK, KV_LEN), lambda b: (b, 0, 0)),
            pl.BlockSpec(
                (B_BLK, QK_ROPE_HEAD_DIM, KV_LEN),
                lambda b: (b, KV_LORA_RANK // QK_ROPE_HEAD_DIM, 0),
            ),
        ],
        out_specs=pl.BlockSpec((B_BLK, H, KV_LORA_RANK), lambda b: (b, 0, 0)),
        out_shape=jax.ShapeDtypeStruct((B, H, KV_LORA_RANK), jnp.float32),
        compiler_params=pltpu.CompilerParams(
            dimension_semantics=("arbitrary",),
        ),
    )(q_nope, q_pe, kv_t, kv_t)

    out = pl.pallas_call(
        _proj_kernel,
        grid=(N_COL_CHUNKS,),
        in_specs=[
            pl.BlockSpec((B, H, KV_LORA_RANK), lambda c: (0, 0, 0)),
            pl.BlockSpec((H, KV_LORA_RANK, V_HEAD_DIM), lambda c: (0, 0, 0)),
            pl.BlockSpec((H * V_HEAD_DIM, COL_CHUNK), lambda c: (0, c)),
        ],
        out_specs=pl.BlockSpec((B, COL_CHUNK), lambda c: (0, c)),
        out_shape=jax.ShapeDtypeStruct((B, D_MODEL), jnp.float32),
        compiler_params=pltpu.CompilerParams(
            dimension_semantics=("arbitrary",),
        ),
    )(o_lat, w_uv, w_o)
    return out


# speedup vs baseline: 1.0052x; 1.0052x over previous
import jax
import jax.numpy as jnp
import numpy as np
from jax.experimental import pallas as pl
from jax.experimental.pallas import tpu as pltpu

B = 32
H = 16
KV_LEN = 4096
KV_LORA_RANK = 512
QK_ROPE_HEAD_DIM = 64
V_HEAD_DIM = 128
D_MODEL = 4096
D_LAT = KV_LORA_RANK + QK_ROPE_HEAD_DIM
SCALE = 1.0 / np.sqrt(128.0 + 64.0)

B_BLK = 2
N_COL_CHUNKS = 2
COL_CHUNK = D_MODEL // N_COL_CHUNKS


def _attn_kernel(qn_ref, qp_ref, lat_ref, rope_ref, o_lat_ref):
    qn = qn_ref[...].astype(jnp.bfloat16)
    qp = qp_ref[...].astype(jnp.bfloat16)
    lat = lat_ref[...].astype(jnp.bfloat16)
    rope = rope_ref[...].astype(jnp.bfloat16)

    s = jax.lax.dot_general(
        qn, lat, (((2,), (1,)), ((0,), (0,))),
        preferred_element_type=jnp.float32,
    ) + jax.lax.dot_general(
        qp, rope, (((2,), (1,)), ((0,), (0,))),
        preferred_element_type=jnp.float32,
    )
    s = s * SCALE
    m = jnp.max(s, axis=-1, keepdims=True)
    p_f32 = jnp.exp(s - m)
    p = p_f32.astype(jnp.bfloat16)
    denom = jnp.sum(p_f32, axis=-1, keepdims=True)

    o_lat_ref[...] = jax.lax.dot_general(
        p, lat, (((2,), (2,)), ((0,), (0,))),
        preferred_element_type=jnp.float32,
    ) / denom


def _proj_kernel(o_lat_ref, w_uv_ref, w_o_ref, out_ref):
    o = jax.lax.dot_general(
        o_lat_ref[...].astype(jnp.bfloat16),
        w_uv_ref[...].astype(jnp.bfloat16),
        (((2,), (1,)), ((1,), (0,))),
        preferred_element_type=jnp.float32,
    )
    o = o.transpose(1, 0, 2).reshape(B, H * V_HEAD_DIM).astype(jnp.bfloat16)
    out_ref[...] = jax.lax.dot_general(
        o, w_o_ref[...].astype(jnp.bfloat16),
        (((1,), (0,)), ((), ())),
        preferred_element_type=jnp.float32,
    )


@jax.jit
def kernel(q_nope, q_pe, kv_cache, w_uv, w_o):
    kv_t = jnp.transpose(kv_cache, (0, 2, 1))

    o_lat = pl.pallas_call(
        _attn_kernel,
        grid=(B // B_BLK,),
        in_specs=[
            pl.BlockSpec((B_BLK, H, KV_LORA_RANK), lambda b: (b, 0, 0)),
            pl.BlockSpec((B_BLK, H, QK_ROPE_HEAD_DIM), lambda b: (b, 0, 0)),
            pl.BlockSpec((B_BLK, KV_LORA_RANK, KV_LEN), lambda b: (b, 0, 0)),
            pl.BlockSpec(
                (B_BLK, QK_ROPE_HEAD_DIM, KV_LEN),
                lambda b: (b, KV_LORA_RANK // QK_ROPE_HEAD_DIM, 0),
            ),
        ],
        out_specs=pl.BlockSpec((B_BLK, H, KV_LORA_RANK), lambda b: (b, 0, 0)),
        out_shape=jax.ShapeDtypeStruct((B, H, KV_LORA_RANK), jnp.float32),
        compiler_params=pltpu.CompilerParams(
            dimension_semantics=("parallel",),
        ),
    )(q_nope, q_pe, kv_t, kv_t)

    out = pl.pallas_call(
        _proj_kernel,
        grid=(N_COL_CHUNKS,),
        in_specs=[
            pl.BlockSpec((B, H, KV_LORA_RANK), lambda c: (0, 0, 0)),
            pl.BlockSpec((H, KV_LORA_RANK, V_HEAD_DIM), lambda c: (0, 0, 0)),
            pl.BlockSpec((H * V_HEAD_DIM, COL_CHUNK), lambda c: (0, c)),
        ],
        out_specs=pl.BlockSpec((B, COL_CHUNK), lambda c: (0, c)),
        out_shape=jax.ShapeDtypeStruct((B, D_MODEL), jnp.float32),
        compiler_params=pltpu.CompilerParams(
            dimension_semantics=("parallel",),
        ),
    )(o_lat, w_uv, w_o)
    return out
